# baseline (speedup 1.0000x reference)
import jax
import jax.numpy as jnp
from jax.experimental import pallas as pl
from jax.experimental.pallas import tpu as pltpu


def _copy_body(img_in, txt_in, img_out, txt_out, sem_img, sem_txt):
    ci = pltpu.make_async_copy(img_in, img_out, sem_img)
    ct = pltpu.make_async_copy(txt_in, txt_out, sem_txt)
    ci.start()
    ct.start()
    ci.wait()
    ct.wait()


def kernel(image_feat, text_feat):
    out = pl.pallas_call(
        _copy_body,
        in_specs=[
            pl.BlockSpec(memory_space=pl.MemorySpace.ANY),
            pl.BlockSpec(memory_space=pl.MemorySpace.ANY),
        ],
        out_specs=[
            pl.BlockSpec(memory_space=pl.MemorySpace.ANY),
            pl.BlockSpec(memory_space=pl.MemorySpace.ANY),
        ],
        out_shape=[
            jax.ShapeDtypeStruct(image_feat.shape, image_feat.dtype),
            jax.ShapeDtypeStruct(text_feat.shape, text_feat.dtype),
        ],
        scratch_shapes=[pltpu.SemaphoreType.DMA, pltpu.SemaphoreType.DMA],
    )(image_feat, text_feat)
    return (out[0], out[1])
